# baseline (speedup 1.0000x reference)
import jax
import jax.numpy as jnp
from jax.experimental import pallas as pl
from jax.experimental.pallas import tpu as pltpu

_LANE = 128
_NB = 16
_HW1 = 20
_PIX1 = 32 * _HW1


def _conv_stack_kernel(x_ref, csel_ref, w1_ref, b1_ref, s2_ref, w2_ref, b2_ref,
                       s3_ref, w3_ref, b3_ref, o_ref, p1s, h1s, z2, h2s, z3):
    f32 = jnp.float32
    bf16 = jnp.bfloat16
    nb = o_ref.shape[0]

    for i in range(nb):
        xi = x_ref[i]
        slabs = [xi[:, 4 * oy:4 * oy + 8, :].reshape(32, 84) for oy in range(20)]
        slab = jnp.concatenate(slabs, axis=0)
        pc = jnp.dot(slab, csel_ref[...],
                     preferred_element_type=f32).astype(bf16)
        for oy in range(20):
            for dx in range(8):
                p1s[i * 256 + dx * 32:i * 256 + (dx + 1) * 32,
                    oy * 32:(oy + 1) * 32] = pc[oy * 32:(oy + 1) * 32,
                                                dx * 32:(dx + 1) * 32]

    for i in range(nb):
        h1 = jnp.dot(w1_ref[...], p1s[i * 256:(i + 1) * 256, :],
                     preferred_element_type=f32)
        h1s[i * 32:(i + 1) * 32, :] = jnp.maximum(h1 + b1_ref[...], 0.0).astype(bf16)

    g2 = jnp.dot(h1s[...], s2_ref[...], preferred_element_type=f32).astype(bf16)
    t2 = s2_ref.shape[1] // _LANE
    cin2 = w2_ref.shape[1] // t2
    for i in range(nb):
        for t in range(t2):
            z2[t * cin2:(t + 1) * cin2, i * _LANE:(i + 1) * _LANE] = \
                g2[i * cin2:(i + 1) * cin2, t * _LANE:(t + 1) * _LANE]
    h2 = jnp.dot(w2_ref[...], z2[...], preferred_element_type=f32)
    h2 = jnp.maximum(h2 + b2_ref[...], 0.0).astype(bf16)

    for i in range(nb):
        h2s[i * 64:(i + 1) * 64, :] = h2[:, i * _LANE:(i + 1) * _LANE]
    g3 = jnp.dot(h2s[...], s3_ref[...], preferred_element_type=f32).astype(bf16)
    t3 = s3_ref.shape[1] // _LANE
    cin3 = w3_ref.shape[1] // t3
    for i in range(nb):
        for t in range(t3):
            z3[t * cin3:(t + 1) * cin3, i * _LANE:(i + 1) * _LANE] = \
                g3[i * cin3:(i + 1) * cin3, t * _LANE:(t + 1) * _LANE]
    h3 = jnp.dot(w3_ref[...], z3[...], preferred_element_type=f32)
    h3 = jnp.maximum(h3 + b3_ref[...], 0.0)

    m3 = o_ref.shape[2]
    for i in range(nb):
        o_ref[i] = h3[:, i * _LANE:i * _LANE + m3].astype(o_ref.dtype)


def _conv_stack(x, csel, w1, b1, s2, w2, b2, s3, w3, b3, nb):
    n = x.shape[0]
    c3 = w3.shape[0]
    m3 = 49
    return pl.pallas_call(
        _conv_stack_kernel,
        out_shape=jax.ShapeDtypeStruct((n, c3, m3), jnp.bfloat16),
        grid=(n // nb,),
        in_specs=[
            pl.BlockSpec((nb,) + x.shape[1:], lambda i: (i, 0, 0, 0)),
            pl.BlockSpec(csel.shape, lambda i: (0, 0)),
            pl.BlockSpec(w1.shape, lambda i: (0, 0)),
            pl.BlockSpec(b1.shape, lambda i: (0, 0)),
            pl.BlockSpec(s2.shape, lambda i: (0, 0)),
            pl.BlockSpec(w2.shape, lambda i: (0, 0)),
            pl.BlockSpec(b2.shape, lambda i: (0, 0)),
            pl.BlockSpec(s3.shape, lambda i: (0, 0)),
            pl.BlockSpec(w3.shape, lambda i: (0, 0)),
            pl.BlockSpec(b3.shape, lambda i: (0, 0)),
        ],
        out_specs=pl.BlockSpec((nb, c3, m3), lambda i: (i, 0, 0)),
        scratch_shapes=[
            pltpu.VMEM((nb * 256, _PIX1), jnp.bfloat16),
            pltpu.VMEM((nb * 32, _PIX1), jnp.bfloat16),
            pltpu.VMEM((w2.shape[1], nb * _LANE), jnp.bfloat16),
            pltpu.VMEM((nb * 64, _LANE), jnp.bfloat16),
            pltpu.VMEM((w3.shape[1], nb * _LANE), jnp.bfloat16),
        ],
        compiler_params=pltpu.CompilerParams(
            dimension_semantics=("parallel",),
            vmem_limit_bytes=100 << 20,
        ),
    )(x, csel, w1, b1, s2, w2, b2, s3, w3, b3)


def _fc_kernel(x_ref, w1_ref, b1_ref, w2_ref, b2_ref, o_ref):
    f32 = jnp.float32
    nc = x_ref.shape[1]
    kp = x_ref.shape[2]
    acc = jnp.zeros((x_ref.shape[0], w1_ref.shape[1]), f32)
    for c in range(nc):
        acc = acc + jnp.dot(x_ref[:, c, :], w1_ref[c * kp:(c + 1) * kp, :],
                            preferred_element_type=f32)
    h = jnp.maximum(acc + b1_ref[...], 0.0).astype(jnp.bfloat16)
    y = jnp.dot(h, w2_ref[...], preferred_element_type=f32) + b2_ref[...]
    o_ref[...] = y.astype(o_ref.dtype)


def _fc(x3, w1, b1, w2, b2, gm):
    m = x3.shape[0]
    n2 = w2.shape[1]
    return pl.pallas_call(
        _fc_kernel,
        out_shape=jax.ShapeDtypeStruct((m, n2), jnp.float32),
        grid=(gm,),
        in_specs=[
            pl.BlockSpec((m // gm, x3.shape[1], x3.shape[2]), lambda i: (i, 0, 0)),
            pl.BlockSpec(w1.shape, lambda i: (0, 0)),
            pl.BlockSpec(b1.shape, lambda i: (0, 0)),
            pl.BlockSpec(w2.shape, lambda i: (0, 0)),
            pl.BlockSpec(b2.shape, lambda i: (0, 0)),
        ],
        out_specs=pl.BlockSpec((m // gm, n2), lambda i: (i, 0)),
        compiler_params=pltpu.CompilerParams(
            dimension_semantics=("parallel",),
            vmem_limit_bytes=64 << 20,
        ),
    )(x3, w1, b1, w2, b2)


def kernel(w1, b1, s2, w2, b2, s3, w3, b3, fc1_w, fc1_b, fc2_w, fc2_b, x):
    bf16 = jnp.bfloat16
    n = x.shape[0]
    nb = next(v for v in (_NB, 4, 2, 1) if n % v == 0)

    dxs = jnp.arange(8)[:, None]
    oxs = jnp.arange(32)[None, :]
    src = 4 * oxs + dxs
    valid = jnp.broadcast_to(oxs < _HW1, (8, 32)).reshape(1, -1)
    csel = (jax.nn.one_hot(src.reshape(-1), 84, axis=0, dtype=jnp.float32)
            * valid).astype(bf16)

    w1p = jnp.transpose(w1.reshape(-1, 4, 8, 8), (0, 3, 1, 2)).reshape(w1.shape)
    oy2 = jnp.arange(_PIX1) // 32
    ox2 = jnp.arange(_PIX1) % 32
    rmap = jax.nn.one_hot(oy2 * _HW1 + ox2, s2.shape[0], axis=1,
                          dtype=jnp.float32) * (ox2 < _HW1)[:, None]
    s2p = jnp.dot(rmap, s2)

    h = _conv_stack(x.astype(bf16), csel, w1p.astype(bf16), b1, s2p.astype(bf16),
                    w2.astype(bf16), b2, s3.astype(bf16), w3.astype(bf16),
                    b3, nb)
    return _fc(h, fc1_w.astype(bf16), fc1_b, fc2_w.astype(bf16), fc2_b,
               gm=2 if n % 2 == 0 else 1)


# speedup vs baseline: 1.0524x; 1.0524x over previous
import jax
import jax.numpy as jnp
from jax.experimental import pallas as pl
from jax.experimental.pallas import tpu as pltpu

_LANE = 128
_NB = 32
_HW1 = 20
_PIX1 = 32 * _HW1


def _conv_stack_kernel(x_ref, csel_ref, w1_ref, b1_ref, s2_ref, w2_ref, b2_ref,
                       s3_ref, w3_ref, b3_ref, o_ref, p1s, h1s, z2, h2s, z3):
    f32 = jnp.float32
    bf16 = jnp.bfloat16
    nb = o_ref.shape[0]

    for i in range(nb):
        xi = x_ref[i].astype(bf16)
        slabs = [xi[:, 4 * oy:4 * oy + 8, :].reshape(32, 84) for oy in range(20)]
        slab = jnp.concatenate(slabs, axis=0)
        pc = jnp.dot(slab, csel_ref[...],
                     preferred_element_type=f32).astype(bf16)
        for oy in range(20):
            for dx in range(8):
                p1s[i * 256 + dx * 32:i * 256 + (dx + 1) * 32,
                    oy * 32:(oy + 1) * 32] = pc[oy * 32:(oy + 1) * 32,
                                                dx * 32:(dx + 1) * 32]

    for i in range(nb):
        h1 = jnp.dot(w1_ref[...], p1s[i * 256:(i + 1) * 256, :],
                     preferred_element_type=f32)
        h1s[i * 32:(i + 1) * 32, :] = jnp.maximum(h1 + b1_ref[...], 0.0).astype(bf16)

    g2 = jnp.dot(h1s[...], s2_ref[...], preferred_element_type=f32).astype(bf16)
    t2 = s2_ref.shape[1] // _LANE
    cin2 = w2_ref.shape[1] // t2
    for i in range(nb):
        for t in range(t2):
            z2[t * cin2:(t + 1) * cin2, i * _LANE:(i + 1) * _LANE] = \
                g2[i * cin2:(i + 1) * cin2, t * _LANE:(t + 1) * _LANE]
    h2 = jnp.dot(w2_ref[...], z2[...], preferred_element_type=f32)
    h2 = jnp.maximum(h2 + b2_ref[...], 0.0).astype(bf16)

    for i in range(nb):
        h2s[i * 64:(i + 1) * 64, :] = h2[:, i * _LANE:(i + 1) * _LANE]
    g3 = jnp.dot(h2s[...], s3_ref[...], preferred_element_type=f32).astype(bf16)
    t3 = s3_ref.shape[1] // _LANE
    cin3 = w3_ref.shape[1] // t3
    for i in range(nb):
        for t in range(t3):
            z3[t * cin3:(t + 1) * cin3, i * _LANE:(i + 1) * _LANE] = \
                g3[i * cin3:(i + 1) * cin3, t * _LANE:(t + 1) * _LANE]
    h3 = jnp.dot(w3_ref[...], z3[...], preferred_element_type=f32)
    h3 = jnp.maximum(h3 + b3_ref[...], 0.0)

    m3 = o_ref.shape[2]
    for i in range(nb):
        o_ref[i] = h3[:, i * _LANE:i * _LANE + m3].astype(o_ref.dtype)


def _conv_stack(x, csel, w1, b1, s2, w2, b2, s3, w3, b3, nb):
    n = x.shape[0]
    c3 = w3.shape[0]
    m3 = 49
    return pl.pallas_call(
        _conv_stack_kernel,
        out_shape=jax.ShapeDtypeStruct((n, c3, m3), jnp.bfloat16),
        grid=(n // nb,),
        in_specs=[
            pl.BlockSpec((nb,) + x.shape[1:], lambda i: (i, 0, 0, 0)),
            pl.BlockSpec(csel.shape, lambda i: (0, 0)),
            pl.BlockSpec(w1.shape, lambda i: (0, 0)),
            pl.BlockSpec(b1.shape, lambda i: (0, 0)),
            pl.BlockSpec(s2.shape, lambda i: (0, 0)),
            pl.BlockSpec(w2.shape, lambda i: (0, 0)),
            pl.BlockSpec(b2.shape, lambda i: (0, 0)),
            pl.BlockSpec(s3.shape, lambda i: (0, 0)),
            pl.BlockSpec(w3.shape, lambda i: (0, 0)),
            pl.BlockSpec(b3.shape, lambda i: (0, 0)),
        ],
        out_specs=pl.BlockSpec((nb, c3, m3), lambda i: (i, 0, 0)),
        scratch_shapes=[
            pltpu.VMEM((nb * 256, _PIX1), jnp.bfloat16),
            pltpu.VMEM((nb * 32, _PIX1), jnp.bfloat16),
            pltpu.VMEM((w2.shape[1], nb * _LANE), jnp.bfloat16),
            pltpu.VMEM((nb * 64, _LANE), jnp.bfloat16),
            pltpu.VMEM((w3.shape[1], nb * _LANE), jnp.bfloat16),
        ],
        compiler_params=pltpu.CompilerParams(
            dimension_semantics=("parallel",),
            vmem_limit_bytes=100 << 20,
        ),
    )(x, csel, w1, b1, s2, w2, b2, s3, w3, b3)


def _fc_kernel(x_ref, w1_ref, b1_ref, w2_ref, b2_ref, o_ref):
    f32 = jnp.float32
    nc = x_ref.shape[1]
    kp = x_ref.shape[2]
    acc = jnp.zeros((x_ref.shape[0], w1_ref.shape[1]), f32)
    for c in range(nc):
        acc = acc + jnp.dot(x_ref[:, c, :], w1_ref[c * kp:(c + 1) * kp, :],
                            preferred_element_type=f32)
    h = jnp.maximum(acc + b1_ref[...], 0.0).astype(jnp.bfloat16)
    y = jnp.dot(h, w2_ref[...], preferred_element_type=f32) + b2_ref[...]
    o_ref[...] = y.astype(o_ref.dtype)


def _fc(x3, w1, b1, w2, b2, gm):
    m = x3.shape[0]
    n2 = w2.shape[1]
    return pl.pallas_call(
        _fc_kernel,
        out_shape=jax.ShapeDtypeStruct((m, n2), jnp.float32),
        grid=(gm,),
        in_specs=[
            pl.BlockSpec((m // gm, x3.shape[1], x3.shape[2]), lambda i: (i, 0, 0)),
            pl.BlockSpec(w1.shape, lambda i: (0, 0)),
            pl.BlockSpec(b1.shape, lambda i: (0, 0)),
            pl.BlockSpec(w2.shape, lambda i: (0, 0)),
            pl.BlockSpec(b2.shape, lambda i: (0, 0)),
        ],
        out_specs=pl.BlockSpec((m // gm, n2), lambda i: (i, 0)),
        compiler_params=pltpu.CompilerParams(
            dimension_semantics=("parallel",),
            vmem_limit_bytes=64 << 20,
        ),
    )(x3, w1, b1, w2, b2)


def kernel(w1, b1, s2, w2, b2, s3, w3, b3, fc1_w, fc1_b, fc2_w, fc2_b, x):
    bf16 = jnp.bfloat16
    n = x.shape[0]
    nb = next(v for v in (_NB, 4, 2, 1) if n % v == 0)

    dxs = jnp.arange(8)[:, None]
    oxs = jnp.arange(32)[None, :]
    src = 4 * oxs + dxs
    valid = jnp.broadcast_to(oxs < _HW1, (8, 32)).reshape(1, -1)
    csel = (jax.nn.one_hot(src.reshape(-1), 84, axis=0, dtype=jnp.float32)
            * valid).astype(bf16)

    w1p = jnp.transpose(w1.reshape(-1, 4, 8, 8), (0, 3, 1, 2)).reshape(w1.shape)
    oy2 = jnp.arange(_PIX1) // 32
    ox2 = jnp.arange(_PIX1) % 32
    rmap = jax.nn.one_hot(oy2 * _HW1 + ox2, s2.shape[0], axis=1,
                          dtype=jnp.float32) * (ox2 < _HW1)[:, None]
    s2p = jnp.dot(rmap, s2)

    h = _conv_stack(x, csel, w1p.astype(bf16), b1, s2p.astype(bf16),
                    w2.astype(bf16), b2, s3.astype(bf16), w3.astype(bf16),
                    b3, nb)
    return _fc(h, fc1_w.astype(bf16), fc1_b, fc2_w.astype(bf16), fc2_b,
               gm=2 if n % 2 == 0 else 1)
